# Initial kernel scaffold; baseline (speedup 1.0000x reference)
import functools
import jax
import jax.numpy as jnp
from jax import lax
from jax.experimental import pallas as pl
from jax.experimental.pallas import tpu as pltpu
from jax.experimental.pallas import tpu_sc as plsc

_N = 10000
_NPAD = 10240
_DIN = 128
_H = 64
_NT = 32
_LANES = 128
_STEPS = 82
_EPAD = _NT * _STEPS * _LANES
_TRASH = 240
_RB = 2048
_GRID = _NPAD // _RB

_mesh = plsc.VectorSubcoreMesh(core_axis_name="c", subcore_axis_name="s")


@functools.partial(
    pl.kernel,
    mesh=_mesh,
    out_type=jax.ShapeDtypeStruct((2, _NPAD), jnp.float32),
    scratch_types=[
        pltpu.VMEM((_STEPS, _LANES), jnp.int32),
        pltpu.VMEM((_NPAD,), jnp.float32),
        pltpu.VMEM_SHARED((_NPAD,), jnp.float32),
    ],
)
def _sc_degree(dst_hbm, out_hbm, idx_d, dloc, dsh):
    cid = lax.axis_index("c")
    sid = lax.axis_index("s")
    wid = cid * 16 + sid
    stripe = _NPAD // 16

    def zero_body(i, _):
        dloc[pl.ds(i * 16, 16)] = jnp.zeros((16,), jnp.float32)
        return 0
    lax.fori_loop(0, _NPAD // 16, zero_body, 0)
    pltpu.sync_copy(dloc.at[pl.ds(sid * stripe, stripe)],
                    dsh.at[pl.ds(sid * stripe, stripe)])
    plsc.subcore_barrier()

    pltpu.sync_copy(dst_hbm.at[wid], idx_d)
    ones = jnp.ones((16,), jnp.float32)

    def acc_body(i, _):
        r = i // (_LANES // 16)
        c = (i % (_LANES // 16)) * 16
        idx16 = idx_d[r, pl.ds(c, 16)]
        plsc.addupdate_scatter(dloc, [idx16], ones)
        return 0
    lax.fori_loop(0, _STEPS * (_LANES // 16), acc_body, 0)

    pltpu.sync_copy(dloc, dsh, add=True)
    plsc.subcore_barrier()
    pltpu.sync_copy(dsh.at[pl.ds(sid * stripe, stripe)],
                    out_hbm.at[cid, pl.ds(sid * stripe, stripe)])


@functools.partial(
    pl.kernel,
    mesh=_mesh,
    out_type=jax.ShapeDtypeStruct((2, _NPAD, _H), jnp.float32),
    scratch_types=[
        pltpu.VMEM((_STEPS, _LANES), jnp.int32),
        pltpu.VMEM((_STEPS, _LANES), jnp.int32),
        pltpu.VMEM((2, _LANES, _H), jnp.float32),
        pltpu.VMEM_SHARED((_NPAD, _H), jnp.float32),
        pltpu.SemaphoreType.DMA,
        pltpu.SemaphoreType.DMA,
    ],
)
def _sc_scatter(h_hbm, src_hbm, dst_hbm, out_hbm,
                idx_s, idx_d, rows, acc_sh, sem0, sem1):
    cid = lax.axis_index("c")
    sid = lax.axis_index("s")
    wid = cid * 16 + sid
    stripe = _NPAD // 16

    def zero_body(i, _):
        r = i // (_H // 16)
        c = (i % (_H // 16)) * 16
        rows[0, r, pl.ds(c, 16)] = jnp.zeros((16,), jnp.float32)
        return 0
    lax.fori_loop(0, _LANES * (_H // 16), zero_body, 0)

    def zcopy_body(i, _):
        pltpu.sync_copy(rows.at[0],
                        acc_sh.at[pl.ds(sid * stripe + i * _LANES, _LANES)])
        return 0
    lax.fori_loop(0, stripe // _LANES, zcopy_body, 0)
    plsc.subcore_barrier()

    pltpu.sync_copy(src_hbm.at[wid], idx_s)
    pltpu.sync_copy(dst_hbm.at[wid], idx_d)

    sems = (sem0, sem1)
    pltpu.async_copy(h_hbm.at[idx_s.at[0]], rows.at[0], sem0)
    pltpu.async_copy(h_hbm.at[idx_s.at[1]], rows.at[1], sem1)

    def outer(o, _):
        for b in range(2):
            j = o * 2 + b
            pltpu.make_async_copy(h_hbm.at[idx_s.at[j]], rows.at[b],
                                  sems[b]).wait()
            pltpu.sync_copy(rows.at[b], acc_sh.at[idx_d.at[j]], add=True)

            @pl.when(j + 2 < _STEPS)
            def _():
                pltpu.async_copy(h_hbm.at[idx_s.at[j + 2]], rows.at[b],
                                 sems[b])
        return 0
    lax.fori_loop(0, _STEPS // 2, outer, 0)

    plsc.subcore_barrier()
    pltpu.sync_copy(acc_sh.at[pl.ds(sid * stripe, stripe)],
                    out_hbm.at[cid, pl.ds(sid * stripe, stripe)])


def _tc1_body(x_ref, w_ref, deg_ref, h_ref, dis_ref):
    deg = deg_ref[0] + deg_ref[1]
    dis = lax.rsqrt(jnp.maximum(deg, 1.0))
    dis_ref[...] = dis
    h = jnp.dot(x_ref[...], w_ref[...], preferred_element_type=jnp.float32)
    h_ref[...] = dis * h


def _tc1(x_pad, W1, deg_col):
    return pl.pallas_call(
        _tc1_body,
        grid=(_GRID,),
        in_specs=[
            pl.BlockSpec((_RB, _DIN), lambda i: (i, 0)),
            pl.BlockSpec((_DIN, _H), lambda i: (0, 0)),
            pl.BlockSpec((2, _RB, 1), lambda i: (0, i, 0)),
        ],
        out_specs=[
            pl.BlockSpec((_RB, _H), lambda i: (i, 0)),
            pl.BlockSpec((_RB, 1), lambda i: (i, 0)),
        ],
        out_shape=[
            jax.ShapeDtypeStruct((_NPAD, _H), jnp.float32),
            jax.ShapeDtypeStruct((_NPAD, 1), jnp.float32),
        ],
    )(x_pad, W1, deg_col)


def _tc2_body(acc_ref, dis_ref, b_ref, w_ref, out_ref):
    i = pl.program_id(0)
    acc = acc_ref[0] + acc_ref[1]
    dis = dis_ref[...]
    z = jnp.maximum(dis * acc + b_ref[...], 0.0)
    row = i * _RB + lax.broadcasted_iota(jnp.int32, (_RB, 1), 0)
    z = jnp.where(row < _N, z, 0.0)
    h2 = jnp.dot(z, w_ref[...], preferred_element_type=jnp.float32)
    out_ref[...] = dis * h2


def _tc2(acc, dis, b1, W2):
    return pl.pallas_call(
        _tc2_body,
        grid=(_GRID,),
        in_specs=[
            pl.BlockSpec((2, _RB, _H), lambda i: (0, i, 0)),
            pl.BlockSpec((_RB, 1), lambda i: (i, 0)),
            pl.BlockSpec((1, _H), lambda i: (0, 0)),
            pl.BlockSpec((_H, _H), lambda i: (0, 0)),
        ],
        out_specs=pl.BlockSpec((_RB, _H), lambda i: (i, 0)),
        out_shape=jax.ShapeDtypeStruct((_NPAD, _H), jnp.float32),
    )(acc, dis, b1, W2)


def _tc3_body(acc_ref, dis_ref, b_ref, lw_ref, lb_ref, out_ref):
    i = pl.program_id(0)
    acc = acc_ref[0] + acc_ref[1]
    z = jnp.maximum(dis_ref[...] * acc + b_ref[...], 0.0)
    row = i * _RB + lax.broadcasted_iota(jnp.int32, (_RB, 1), 0)
    z = jnp.where(row < _N, z, 0.0)
    part = jnp.sum(z, axis=0, keepdims=True) * (1.0 / _N)
    contrib = jnp.dot(part, lw_ref[...], preferred_element_type=jnp.float32)

    @pl.when(i == 0)
    def _():
        out_ref[...] = lb_ref[...]
    out_ref[...] += contrib


def _tc3(acc, dis, b2, lin_W, lin_b):
    return pl.pallas_call(
        _tc3_body,
        grid=(_GRID,),
        in_specs=[
            pl.BlockSpec((2, _RB, _H), lambda i: (0, i, 0)),
            pl.BlockSpec((_RB, 1), lambda i: (i, 0)),
            pl.BlockSpec((1, _H), lambda i: (0, 0)),
            pl.BlockSpec((_H, 2), lambda i: (0, 0)),
            pl.BlockSpec((1, 2), lambda i: (0, 0)),
        ],
        out_specs=pl.BlockSpec((1, 2), lambda i: (0, 0)),
        out_shape=jax.ShapeDtypeStruct((1, 2), jnp.float32),
    )(acc, dis, b2, lin_W, lin_b)


def kernel(x, edge_index, batch, W1, b1, W2, b2, lin_W, lin_b):
    n = x.shape[0]
    x_pad = jnp.pad(x, ((0, _NPAD - n), (0, 0)))
    loop = jnp.arange(n, dtype=jnp.int32)
    npad_e = _EPAD - edge_index.shape[1] - n
    pad_idx = _N + (jnp.arange(npad_e, dtype=jnp.int32) % _TRASH)
    src = jnp.concatenate([edge_index[0], loop, pad_idx])
    dst = jnp.concatenate([edge_index[1], loop, pad_idx])
    src3 = src.reshape(_NT, _STEPS, _LANES)
    dst3 = dst.reshape(_NT, _STEPS, _LANES)

    deg2 = _sc_degree(dst3)
    deg_col = deg2.reshape(2, _NPAD, 1)
    h1p, dis = _tc1(x_pad, W1, deg_col)
    acc1 = _sc_scatter(h1p, src3, dst3)
    h2p = _tc2(acc1, dis, b1.reshape(1, _H), W2)
    acc2 = _sc_scatter(h2p, src3, dst3)
    return _tc3(acc2, dis, b2.reshape(1, _H), lin_W, lin_b.reshape(1, 2))


# speedup vs baseline: 38.3213x; 38.3213x over previous
import functools
import jax
import jax.numpy as jnp
from jax import lax
from jax.experimental import pallas as pl
from jax.experimental.pallas import tpu as pltpu
from jax.experimental.pallas import tpu_sc as plsc

_N = 10000
_NPAD = 10240
_DIN = 128
_H = 64
_NT = 32
_LANES = 128
_STEPS = 82
_EPAD = _NT * _STEPS * _LANES
_TRASH = 240
_RB = 2048
_GRID = _NPAD // _RB

_mesh = plsc.VectorSubcoreMesh(core_axis_name="c", subcore_axis_name="s")


@functools.partial(
    pl.kernel,
    mesh=_mesh,
    out_type=jax.ShapeDtypeStruct((2, _NPAD), jnp.float32),
    scratch_types=[
        pltpu.VMEM((_STEPS, _LANES), jnp.int32),
        pltpu.VMEM((_NPAD // 16,), jnp.float32),
        pltpu.VMEM((_LANES,), jnp.float32),
        pltpu.VMEM_SHARED((_NPAD,), jnp.float32),
    ],
    compiler_params=pltpu.CompilerParams(use_tc_tiling_on_sc=False),
)
def _sc_degree(dst_hbm, out_hbm, idx_d, zbuf, obuf, dsh):
    cid = lax.axis_index("c")
    sid = lax.axis_index("s")
    wid = cid * 16 + sid
    stripe = _NPAD // 16

    def zero_body(i, _):
        zbuf[pl.ds(i * 16, 16)] = jnp.zeros((16,), jnp.float32)
        return 0
    lax.fori_loop(0, stripe // 16, zero_body, 0)

    def ones_body(i, _):
        obuf[pl.ds(i * 16, 16)] = jnp.ones((16,), jnp.float32)
        return 0
    lax.fori_loop(0, _LANES // 16, ones_body, 0)

    pltpu.sync_copy(zbuf, dsh.at[pl.ds(sid * stripe, stripe)])
    plsc.subcore_barrier()

    pltpu.sync_copy(dst_hbm.at[wid], idx_d)

    def acc_body(j, _):
        pltpu.sync_copy(obuf, dsh.at[idx_d.at[j]], add=True)
        return 0
    lax.fori_loop(0, _STEPS, acc_body, 0)

    plsc.subcore_barrier()
    pltpu.sync_copy(dsh.at[pl.ds(sid * stripe, stripe)],
                    out_hbm.at[cid, pl.ds(sid * stripe, stripe)])


@functools.partial(
    pl.kernel,
    mesh=_mesh,
    out_type=jax.ShapeDtypeStruct((2, _NPAD, _H), jnp.float32),
    scratch_types=[
        pltpu.VMEM((_STEPS, _LANES), jnp.int32),
        pltpu.VMEM((_STEPS, _LANES), jnp.int32),
        pltpu.VMEM((2, _LANES, _H), jnp.float32),
        pltpu.VMEM_SHARED((_NPAD, _H), jnp.float32),
        pltpu.SemaphoreType.DMA,
        pltpu.SemaphoreType.DMA,
    ],
    compiler_params=pltpu.CompilerParams(use_tc_tiling_on_sc=False),
)
def _sc_scatter(h_hbm, src_hbm, dst_hbm, out_hbm,
                idx_s, idx_d, rows, acc_sh, sem0, sem1):
    cid = lax.axis_index("c")
    sid = lax.axis_index("s")
    wid = cid * 16 + sid
    stripe = _NPAD // 16

    def zero_body(i, _):
        r = i // (_H // 16)
        c = (i % (_H // 16)) * 16
        rows[0, r, pl.ds(c, 16)] = jnp.zeros((16,), jnp.float32)
        return 0
    lax.fori_loop(0, _LANES * (_H // 16), zero_body, 0)

    def zcopy_body(i, _):
        pltpu.sync_copy(rows.at[0],
                        acc_sh.at[pl.ds(sid * stripe + i * _LANES, _LANES)])
        return 0
    lax.fori_loop(0, stripe // _LANES, zcopy_body, 0)
    plsc.subcore_barrier()

    pltpu.sync_copy(src_hbm.at[wid], idx_s)
    pltpu.sync_copy(dst_hbm.at[wid], idx_d)

    sems = (sem0, sem1)
    pltpu.async_copy(h_hbm.at[idx_s.at[0]], rows.at[0], sem0)
    pltpu.async_copy(h_hbm.at[idx_s.at[1]], rows.at[1], sem1)

    def outer(o, _):
        for b in range(2):
            j = o * 2 + b
            pltpu.make_async_copy(h_hbm.at[idx_s.at[j]], rows.at[b],
                                  sems[b]).wait()
            pltpu.sync_copy(rows.at[b], acc_sh.at[idx_d.at[j]], add=True)

            @pl.when(j + 2 < _STEPS)
            def _():
                pltpu.async_copy(h_hbm.at[idx_s.at[j + 2]], rows.at[b],
                                 sems[b])
        return 0
    lax.fori_loop(0, _STEPS // 2, outer, 0)

    plsc.subcore_barrier()
    pltpu.sync_copy(acc_sh.at[pl.ds(sid * stripe, stripe)],
                    out_hbm.at[cid, pl.ds(sid * stripe, stripe)])


def _tc1_body(x_ref, w_ref, deg_ref, h_ref, dis_ref):
    deg = deg_ref[0] + deg_ref[1]
    dis = lax.rsqrt(jnp.maximum(deg, 1.0))
    dis_ref[...] = dis
    h = jnp.dot(x_ref[...], w_ref[...], preferred_element_type=jnp.float32)
    h_ref[...] = dis * h


def _tc1(x_pad, W1, deg_col):
    return pl.pallas_call(
        _tc1_body,
        grid=(_GRID,),
        in_specs=[
            pl.BlockSpec((_RB, _DIN), lambda i: (i, 0)),
            pl.BlockSpec((_DIN, _H), lambda i: (0, 0)),
            pl.BlockSpec((2, _RB, 1), lambda i: (0, i, 0)),
        ],
        out_specs=[
            pl.BlockSpec((_RB, _H), lambda i: (i, 0)),
            pl.BlockSpec((_RB, 1), lambda i: (i, 0)),
        ],
        out_shape=[
            jax.ShapeDtypeStruct((_NPAD, _H), jnp.float32),
            jax.ShapeDtypeStruct((_NPAD, 1), jnp.float32),
        ],
    )(x_pad, W1, deg_col)


def _tc2_body(acc_ref, dis_ref, b_ref, w_ref, out_ref):
    i = pl.program_id(0)
    acc = acc_ref[0] + acc_ref[1]
    dis = dis_ref[...]
    z = jnp.maximum(dis * acc + b_ref[...], 0.0)
    row = i * _RB + lax.broadcasted_iota(jnp.int32, (_RB, 1), 0)
    z = jnp.where(row < _N, z, 0.0)
    h2 = jnp.dot(z, w_ref[...], preferred_element_type=jnp.float32)
    out_ref[...] = dis * h2


def _tc2(acc, dis, b1, W2):
    return pl.pallas_call(
        _tc2_body,
        grid=(_GRID,),
        in_specs=[
            pl.BlockSpec((2, _RB, _H), lambda i: (0, i, 0)),
            pl.BlockSpec((_RB, 1), lambda i: (i, 0)),
            pl.BlockSpec((1, _H), lambda i: (0, 0)),
            pl.BlockSpec((_H, _H), lambda i: (0, 0)),
        ],
        out_specs=pl.BlockSpec((_RB, _H), lambda i: (i, 0)),
        out_shape=jax.ShapeDtypeStruct((_NPAD, _H), jnp.float32),
    )(acc, dis, b1, W2)


def _tc3_body(acc_ref, dis_ref, b_ref, lw_ref, lb_ref, out_ref):
    i = pl.program_id(0)
    acc = acc_ref[0] + acc_ref[1]
    z = jnp.maximum(dis_ref[...] * acc + b_ref[...], 0.0)
    row = i * _RB + lax.broadcasted_iota(jnp.int32, (_RB, 1), 0)
    z = jnp.where(row < _N, z, 0.0)
    part = jnp.sum(z, axis=0, keepdims=True) * (1.0 / _N)
    contrib = jnp.dot(part, lw_ref[...], preferred_element_type=jnp.float32)

    @pl.when(i == 0)
    def _():
        out_ref[...] = lb_ref[...]
    out_ref[...] += contrib


def _tc3(acc, dis, b2, lin_W, lin_b):
    return pl.pallas_call(
        _tc3_body,
        grid=(_GRID,),
        in_specs=[
            pl.BlockSpec((2, _RB, _H), lambda i: (0, i, 0)),
            pl.BlockSpec((_RB, 1), lambda i: (i, 0)),
            pl.BlockSpec((1, _H), lambda i: (0, 0)),
            pl.BlockSpec((_H, 2), lambda i: (0, 0)),
            pl.BlockSpec((1, 2), lambda i: (0, 0)),
        ],
        out_specs=pl.BlockSpec((1, 2), lambda i: (0, 0)),
        out_shape=jax.ShapeDtypeStruct((1, 2), jnp.float32),
    )(acc, dis, b2, lin_W, lin_b)


def kernel(x, edge_index, batch, W1, b1, W2, b2, lin_W, lin_b):
    n = x.shape[0]
    x_pad = jnp.pad(x, ((0, _NPAD - n), (0, 0)))
    loop = jnp.arange(n, dtype=jnp.int32)
    npad_e = _EPAD - edge_index.shape[1] - n
    pad_idx = _N + (jnp.arange(npad_e, dtype=jnp.int32) % _TRASH)
    src = jnp.concatenate([edge_index[0], loop, pad_idx])
    dst = jnp.concatenate([edge_index[1], loop, pad_idx])
    src3 = src.reshape(_NT, _STEPS, _LANES)
    dst3 = dst.reshape(_NT, _STEPS, _LANES)

    deg2 = _sc_degree(dst3)
    deg_col = deg2.reshape(2, _NPAD, 1)
    h1p, dis = _tc1(x_pad, W1, deg_col)
    acc1 = _sc_scatter(h1p, src3, dst3)
    h2p = _tc2(acc1, dis, b1.reshape(1, _H), W2)
    acc2 = _sc_scatter(h2p, src3, dst3)
    return _tc3(acc2, dis, b2.reshape(1, _H), lin_W, lin_b.reshape(1, 2))


# speedup vs baseline: 41.3470x; 1.0790x over previous
import functools
import jax
import jax.numpy as jnp
from jax import lax
from jax.experimental import pallas as pl
from jax.experimental.pallas import tpu as pltpu
from jax.experimental.pallas import tpu_sc as plsc

_N = 10000
_NPAD = 10240
_DIN = 128
_H = 64
_NT = 32
_LANES = 128
_STEPS = 84
_EPAD = _NT * _STEPS * _LANES
_TRASH = 240
_RB = 2048
_GRID = _NPAD // _RB

_mesh = plsc.VectorSubcoreMesh(core_axis_name="c", subcore_axis_name="s")


@functools.partial(
    pl.kernel,
    mesh=_mesh,
    out_type=jax.ShapeDtypeStruct((2, _NPAD), jnp.float32),
    scratch_types=[
        pltpu.VMEM((_STEPS, _LANES), jnp.int32),
        pltpu.VMEM((_NPAD // 16,), jnp.float32),
        pltpu.VMEM((_LANES,), jnp.float32),
        pltpu.VMEM_SHARED((_NPAD,), jnp.float32),
        pltpu.SemaphoreType.DMA,
    ],
    compiler_params=pltpu.CompilerParams(use_tc_tiling_on_sc=False),
)
def _sc_degree(dst_hbm, out_hbm, idx_d, zbuf, obuf, dsh, dsem):
    cid = lax.axis_index("c")
    sid = lax.axis_index("s")
    wid = cid * 16 + sid
    stripe = _NPAD // 16

    def zero_body(i, _):
        zbuf[pl.ds(i * 16, 16)] = jnp.zeros((16,), jnp.float32)
        return 0
    lax.fori_loop(0, stripe // 16, zero_body, 0)

    def ones_body(i, _):
        obuf[pl.ds(i * 16, 16)] = jnp.ones((16,), jnp.float32)
        return 0
    lax.fori_loop(0, _LANES // 16, ones_body, 0)

    pltpu.sync_copy(zbuf, dsh.at[pl.ds(sid * stripe, stripe)])
    plsc.subcore_barrier()

    pltpu.sync_copy(dst_hbm.at[wid], idx_d)

    def acc_body(j, _):
        pltpu.async_copy(obuf, dsh.at[idx_d.at[j]], dsem, add=True)
        return 0
    lax.fori_loop(0, _STEPS, acc_body, 0)

    def drain_body(j, _):
        pltpu.make_async_copy(obuf, dsh.at[idx_d.at[j]], dsem).wait()
        return 0
    lax.fori_loop(0, _STEPS, drain_body, 0)

    plsc.subcore_barrier()
    pltpu.sync_copy(dsh.at[pl.ds(sid * stripe, stripe)],
                    out_hbm.at[cid, pl.ds(sid * stripe, stripe)])


@functools.partial(
    pl.kernel,
    mesh=_mesh,
    out_type=jax.ShapeDtypeStruct((2, _NPAD, _H), jnp.float32),
    scratch_types=[
        pltpu.VMEM((_STEPS, _LANES), jnp.int32),
        pltpu.VMEM((_STEPS, _LANES), jnp.int32),
        pltpu.VMEM((4, _LANES, _H), jnp.float32),
        pltpu.VMEM_SHARED((_NPAD, _H), jnp.float32),
        [pltpu.SemaphoreType.DMA] * 4,
        [pltpu.SemaphoreType.DMA] * 4,
    ],
    compiler_params=pltpu.CompilerParams(use_tc_tiling_on_sc=False),
)
def _sc_scatter(h_hbm, src_hbm, dst_hbm, out_hbm,
                idx_s, idx_d, rows, acc_sh, gsem, ssem):
    cid = lax.axis_index("c")
    sid = lax.axis_index("s")
    wid = cid * 16 + sid
    stripe = _NPAD // 16

    def zero_body(i, _):
        r = i // (_H // 16)
        c = (i % (_H // 16)) * 16
        rows[0, r, pl.ds(c, 16)] = jnp.zeros((16,), jnp.float32)
        return 0
    lax.fori_loop(0, _LANES * (_H // 16), zero_body, 0)

    def zcopy_body(i, _):
        pltpu.sync_copy(rows.at[0],
                        acc_sh.at[pl.ds(sid * stripe + i * _LANES, _LANES)])
        return 0
    lax.fori_loop(0, stripe // _LANES, zcopy_body, 0)
    plsc.subcore_barrier()

    pltpu.sync_copy(src_hbm.at[wid], idx_s)
    pltpu.sync_copy(dst_hbm.at[wid], idx_d)

    def outer(o, _):
        for i in range(4):
            j = o * 4 + i

            @pl.when(j < _STEPS)
            def _():
                @pl.when(j >= 4)
                def _():
                    pltpu.make_async_copy(
                        rows.at[i], acc_sh.at[idx_d.at[j - 4]],
                        ssem[i]).wait()
                pltpu.async_copy(h_hbm.at[idx_s.at[j]], rows.at[i], gsem[i])

            jj = j - 2
            bs = (i + 2) % 4

            @pl.when((0 <= jj) & (jj < _STEPS))
            def _():
                pltpu.make_async_copy(h_hbm.at[idx_s.at[jj]], rows.at[bs],
                                      gsem[bs]).wait()
                pltpu.async_copy(rows.at[bs], acc_sh.at[idx_d.at[jj]],
                                 ssem[bs], add=True)
        return 0
    lax.fori_loop(0, _STEPS // 4 + 1, outer, 0)

    for b in range(4):
        pltpu.make_async_copy(rows.at[b],
                              acc_sh.at[idx_d.at[_STEPS - 4 + b]],
                              ssem[b]).wait()

    plsc.subcore_barrier()
    pltpu.sync_copy(acc_sh.at[pl.ds(sid * stripe, stripe)],
                    out_hbm.at[cid, pl.ds(sid * stripe, stripe)])


def _tc1_body(x_ref, w_ref, deg_ref, h_ref, dis_ref):
    deg = deg_ref[0] + deg_ref[1]
    dis = lax.rsqrt(jnp.maximum(deg, 1.0))
    dis_ref[...] = dis
    h = jnp.dot(x_ref[...], w_ref[...], preferred_element_type=jnp.float32)
    h_ref[...] = dis * h


def _tc1(x_pad, W1, deg_col):
    return pl.pallas_call(
        _tc1_body,
        grid=(_GRID,),
        in_specs=[
            pl.BlockSpec((_RB, _DIN), lambda i: (i, 0)),
            pl.BlockSpec((_DIN, _H), lambda i: (0, 0)),
            pl.BlockSpec((2, _RB, 1), lambda i: (0, i, 0)),
        ],
        out_specs=[
            pl.BlockSpec((_RB, _H), lambda i: (i, 0)),
            pl.BlockSpec((_RB, 1), lambda i: (i, 0)),
        ],
        out_shape=[
            jax.ShapeDtypeStruct((_NPAD, _H), jnp.float32),
            jax.ShapeDtypeStruct((_NPAD, 1), jnp.float32),
        ],
    )(x_pad, W1, deg_col)


def _tc2_body(acc_ref, dis_ref, b_ref, w_ref, out_ref):
    i = pl.program_id(0)
    acc = acc_ref[0] + acc_ref[1]
    dis = dis_ref[...]
    z = jnp.maximum(dis * acc + b_ref[...], 0.0)
    row = i * _RB + lax.broadcasted_iota(jnp.int32, (_RB, 1), 0)
    z = jnp.where(row < _N, z, 0.0)
    h2 = jnp.dot(z, w_ref[...], preferred_element_type=jnp.float32)
    out_ref[...] = dis * h2


def _tc2(acc, dis, b1, W2):
    return pl.pallas_call(
        _tc2_body,
        grid=(_GRID,),
        in_specs=[
            pl.BlockSpec((2, _RB, _H), lambda i: (0, i, 0)),
            pl.BlockSpec((_RB, 1), lambda i: (i, 0)),
            pl.BlockSpec((1, _H), lambda i: (0, 0)),
            pl.BlockSpec((_H, _H), lambda i: (0, 0)),
        ],
        out_specs=pl.BlockSpec((_RB, _H), lambda i: (i, 0)),
        out_shape=jax.ShapeDtypeStruct((_NPAD, _H), jnp.float32),
    )(acc, dis, b1, W2)


def _tc3_body(acc_ref, dis_ref, b_ref, lw_ref, lb_ref, out_ref):
    i = pl.program_id(0)
    acc = acc_ref[0] + acc_ref[1]
    z = jnp.maximum(dis_ref[...] * acc + b_ref[...], 0.0)
    row = i * _RB + lax.broadcasted_iota(jnp.int32, (_RB, 1), 0)
    z = jnp.where(row < _N, z, 0.0)
    part = jnp.sum(z, axis=0, keepdims=True) * (1.0 / _N)
    contrib = jnp.dot(part, lw_ref[...], preferred_element_type=jnp.float32)

    @pl.when(i == 0)
    def _():
        out_ref[...] = lb_ref[...]
    out_ref[...] += contrib


def _tc3(acc, dis, b2, lin_W, lin_b):
    return pl.pallas_call(
        _tc3_body,
        grid=(_GRID,),
        in_specs=[
            pl.BlockSpec((2, _RB, _H), lambda i: (0, i, 0)),
            pl.BlockSpec((_RB, 1), lambda i: (i, 0)),
            pl.BlockSpec((1, _H), lambda i: (0, 0)),
            pl.BlockSpec((_H, 2), lambda i: (0, 0)),
            pl.BlockSpec((1, 2), lambda i: (0, 0)),
        ],
        out_specs=pl.BlockSpec((1, 2), lambda i: (0, 0)),
        out_shape=jax.ShapeDtypeStruct((1, 2), jnp.float32),
    )(acc, dis, b2, lin_W, lin_b)


def kernel(x, edge_index, batch, W1, b1, W2, b2, lin_W, lin_b):
    n = x.shape[0]
    x_pad = jnp.pad(x, ((0, _NPAD - n), (0, 0)))
    loop = jnp.arange(n, dtype=jnp.int32)
    npad_e = _EPAD - edge_index.shape[1] - n
    pad_idx = _N + (jnp.arange(npad_e, dtype=jnp.int32) % _TRASH)
    src = jnp.concatenate([edge_index[0], loop, pad_idx])
    dst = jnp.concatenate([edge_index[1], loop, pad_idx])
    src3 = src.reshape(_NT, _STEPS, _LANES)
    dst3 = dst.reshape(_NT, _STEPS, _LANES)

    deg2 = _sc_degree(dst3)
    deg_col = deg2.reshape(2, _NPAD, 1)
    h1p, dis = _tc1(x_pad, W1, deg_col)
    acc1 = _sc_scatter(h1p, src3, dst3)
    h2p = _tc2(acc1, dis, b1.reshape(1, _H), W2)
    acc2 = _sc_scatter(h2p, src3, dst3)
    return _tc3(acc2, dis, b2.reshape(1, _H), lin_W, lin_b.reshape(1, 2))


# speedup vs baseline: 45.4234x; 1.0986x over previous
import functools
import jax
import jax.numpy as jnp
from jax import lax
from jax.experimental import pallas as pl
from jax.experimental.pallas import tpu as pltpu
from jax.experimental.pallas import tpu_sc as plsc

_N = 10000
_NACC = 10240
_DIN = 128
_H = 64
_NT = 32
_LANES = 128
_EROWS = 2500
_STEPS = 78
_RB = 2000
_GRID = _N // _RB

_mesh = plsc.VectorSubcoreMesh(core_axis_name="c", subcore_axis_name="s")


@functools.partial(
    pl.kernel,
    mesh=_mesh,
    out_type=jax.ShapeDtypeStruct((2, _NACC), jnp.float32),
    scratch_types=[
        pltpu.VMEM((_STEPS + 1, _LANES), jnp.int32),
        pltpu.VMEM((_NACC // 16,), jnp.float32),
        pltpu.VMEM((_LANES,), jnp.float32),
        pltpu.VMEM_SHARED((_NACC,), jnp.float32),
        pltpu.SemaphoreType.DMA,
    ],
    compiler_params=pltpu.CompilerParams(use_tc_tiling_on_sc=False),
)
def _sc_degree(dst_hbm, out_hbm, idx_d, zbuf, obuf, dsh, dsem):
    cid = lax.axis_index("c")
    sid = lax.axis_index("s")
    wid = cid * 16 + sid
    stripe = _NACC // 16

    def zero_body(i, _):
        zbuf[pl.ds(i * 16, 16)] = jnp.zeros((16,), jnp.float32)
        return 0
    lax.fori_loop(0, stripe // 16, zero_body, 0)

    def ones_body(i, _):
        obuf[pl.ds(i * 16, 16)] = jnp.ones((16,), jnp.float32)
        return 0
    lax.fori_loop(0, _LANES // 16, ones_body, 0)

    pltpu.sync_copy(zbuf, dsh.at[pl.ds(sid * stripe, stripe)])
    plsc.subcore_barrier()

    pltpu.sync_copy(dst_hbm.at[pl.ds(wid * _STEPS, _STEPS)],
                    idx_d.at[pl.ds(0, _STEPS)])

    @pl.when(wid < _EROWS - _NT * _STEPS)
    def _():
        pltpu.sync_copy(dst_hbm.at[_NT * _STEPS + wid], idx_d.at[_STEPS])

    def acc_body(j, _):
        pltpu.async_copy(obuf, dsh.at[idx_d.at[j]], dsem, add=True)
        return 0
    lax.fori_loop(0, _STEPS, acc_body, 0)

    @pl.when(wid < _EROWS - _NT * _STEPS)
    def _():
        pltpu.async_copy(obuf, dsh.at[idx_d.at[_STEPS]], dsem, add=True)

    def drain_body(j, _):
        pltpu.make_async_copy(obuf, dsh.at[idx_d.at[j]], dsem).wait()
        return 0
    lax.fori_loop(0, _STEPS, drain_body, 0)

    @pl.when(wid < _EROWS - _NT * _STEPS)
    def _():
        pltpu.make_async_copy(obuf, dsh.at[idx_d.at[_STEPS]], dsem).wait()

    plsc.subcore_barrier()
    pltpu.sync_copy(dsh.at[pl.ds(sid * stripe, stripe)],
                    out_hbm.at[cid, pl.ds(sid * stripe, stripe)])


@functools.partial(
    pl.kernel,
    mesh=_mesh,
    out_type=jax.ShapeDtypeStruct((2, _N, _H), jnp.float32),
    scratch_types=[
        pltpu.VMEM((_STEPS + 1, _LANES), jnp.int32),
        pltpu.VMEM((_STEPS + 1, _LANES), jnp.int32),
        pltpu.VMEM((4, _LANES, _H), jnp.float32),
        pltpu.VMEM_SHARED((_NACC, _H), jnp.float32),
        [pltpu.SemaphoreType.DMA] * 4,
        [pltpu.SemaphoreType.DMA] * 4,
    ],
    compiler_params=pltpu.CompilerParams(use_tc_tiling_on_sc=False),
)
def _sc_scatter(h_hbm, src_hbm, dst_hbm, out_hbm,
                idx_s, idx_d, rows, acc_sh, gsem, ssem):
    cid = lax.axis_index("c")
    sid = lax.axis_index("s")
    wid = cid * 16 + sid
    zstripe = _NACC // 16
    ostripe = _N // 16
    has_tail = wid < _EROWS - _NT * _STEPS

    def zero_body(i, _):
        r = i // (_H // 16)
        c = (i % (_H // 16)) * 16
        rows[0, r, pl.ds(c, 16)] = jnp.zeros((16,), jnp.float32)
        return 0
    lax.fori_loop(0, _LANES * (_H // 16), zero_body, 0)

    def zcopy_body(i, _):
        pltpu.sync_copy(rows.at[0],
                        acc_sh.at[pl.ds(sid * zstripe + i * _LANES, _LANES)])
        return 0
    lax.fori_loop(0, zstripe // _LANES, zcopy_body, 0)
    plsc.subcore_barrier()

    pltpu.sync_copy(src_hbm.at[pl.ds(wid * _STEPS, _STEPS)],
                    idx_s.at[pl.ds(0, _STEPS)])
    pltpu.sync_copy(dst_hbm.at[pl.ds(wid * _STEPS, _STEPS)],
                    idx_d.at[pl.ds(0, _STEPS)])

    @pl.when(has_tail)
    def _():
        pltpu.sync_copy(src_hbm.at[_NT * _STEPS + wid], idx_s.at[_STEPS])
        pltpu.sync_copy(dst_hbm.at[_NT * _STEPS + wid], idx_d.at[_STEPS])

    def outer(o, _):
        for i in range(4):
            j = o * 4 + i

            @pl.when(j < _STEPS)
            def _():
                @pl.when(j >= 4)
                def _():
                    pltpu.make_async_copy(
                        rows.at[i], acc_sh.at[idx_d.at[j - 4]],
                        ssem[i]).wait()
                pltpu.async_copy(h_hbm.at[idx_s.at[j]], rows.at[i], gsem[i])

            jj = j - 2
            bs = (i + 2) % 4

            @pl.when((0 <= jj) & (jj < _STEPS))
            def _():
                pltpu.make_async_copy(h_hbm.at[idx_s.at[jj]], rows.at[bs],
                                      gsem[bs]).wait()
                pltpu.async_copy(rows.at[bs], acc_sh.at[idx_d.at[jj]],
                                 ssem[bs], add=True)
        return 0
    lax.fori_loop(0, _STEPS // 4 + 1, outer, 0)

    for b in range(4):
        pltpu.make_async_copy(rows.at[b],
                              acc_sh.at[idx_d.at[_STEPS - 4 + (b + 2) % 4]],
                              ssem[b]).wait()

    @pl.when(has_tail)
    def _():
        pltpu.async_copy(h_hbm.at[idx_s.at[_STEPS]], rows.at[0],
                         gsem[0]).wait()
        pltpu.sync_copy(rows.at[0], acc_sh.at[idx_d.at[_STEPS]], add=True)

    plsc.subcore_barrier()
    pltpu.sync_copy(acc_sh.at[pl.ds(sid * ostripe, ostripe)],
                    out_hbm.at[cid, pl.ds(sid * ostripe, ostripe)])


def _tca_body(x_ref, w_ref, h_ref):
    h_ref[...] = jnp.dot(x_ref[...], w_ref[...],
                         preferred_element_type=jnp.float32)


def _tca(x, W1):
    return pl.pallas_call(
        _tca_body,
        grid=(_GRID,),
        in_specs=[
            pl.BlockSpec((_RB, _DIN), lambda i: (i, 0)),
            pl.BlockSpec((_DIN, _H), lambda i: (0, 0)),
        ],
        out_specs=pl.BlockSpec((_RB, _H), lambda i: (i, 0)),
        out_shape=jax.ShapeDtypeStruct((_N, _H), jnp.float32),
    )(x, W1)


def _tcb_body(h_ref, deg_ref, hp_ref, dis_ref):
    deg = deg_ref[0] + deg_ref[1] + 1.0
    dis = lax.rsqrt(deg)
    dis_ref[...] = dis
    hp_ref[...] = dis * h_ref[...]


def _tcb(h1, deg_col):
    return pl.pallas_call(
        _tcb_body,
        grid=(_GRID,),
        in_specs=[
            pl.BlockSpec((_RB, _H), lambda i: (i, 0)),
            pl.BlockSpec((2, _RB, 1), lambda i: (0, i, 0)),
        ],
        out_specs=[
            pl.BlockSpec((_RB, _H), lambda i: (i, 0)),
            pl.BlockSpec((_RB, 1), lambda i: (i, 0)),
        ],
        out_shape=[
            jax.ShapeDtypeStruct((_N, _H), jnp.float32),
            jax.ShapeDtypeStruct((_N, 1), jnp.float32),
        ],
    )(h1, deg_col)


def _tc2_body(acc_ref, hp_ref, dis_ref, b_ref, w_ref, out_ref):
    acc = acc_ref[0] + acc_ref[1] + hp_ref[...]
    dis = dis_ref[...]
    z = jnp.maximum(dis * acc + b_ref[...], 0.0)
    h2 = jnp.dot(z, w_ref[...], preferred_element_type=jnp.float32)
    out_ref[...] = dis * h2


def _tc2(acc, h1p, dis, b1, W2):
    return pl.pallas_call(
        _tc2_body,
        grid=(_GRID,),
        in_specs=[
            pl.BlockSpec((2, _RB, _H), lambda i: (0, i, 0)),
            pl.BlockSpec((_RB, _H), lambda i: (i, 0)),
            pl.BlockSpec((_RB, 1), lambda i: (i, 0)),
            pl.BlockSpec((1, _H), lambda i: (0, 0)),
            pl.BlockSpec((_H, _H), lambda i: (0, 0)),
        ],
        out_specs=pl.BlockSpec((_RB, _H), lambda i: (i, 0)),
        out_shape=jax.ShapeDtypeStruct((_N, _H), jnp.float32),
    )(acc, h1p, dis, b1, W2)


def _tc3_body(acc_ref, hp_ref, dis_ref, b_ref, lw_ref, lb_ref, out_ref):
    i = pl.program_id(0)
    acc = acc_ref[0] + acc_ref[1] + hp_ref[...]
    z = jnp.maximum(dis_ref[...] * acc + b_ref[...], 0.0)
    part = jnp.sum(z, axis=0, keepdims=True) * (1.0 / _N)
    contrib = jnp.dot(part, lw_ref[...], preferred_element_type=jnp.float32)

    @pl.when(i == 0)
    def _():
        out_ref[...] = lb_ref[...]
    out_ref[...] += contrib


def _tc3(acc, h2p, dis, b2, lin_W, lin_b):
    return pl.pallas_call(
        _tc3_body,
        grid=(_GRID,),
        in_specs=[
            pl.BlockSpec((2, _RB, _H), lambda i: (0, i, 0)),
            pl.BlockSpec((_RB, _H), lambda i: (i, 0)),
            pl.BlockSpec((_RB, 1), lambda i: (i, 0)),
            pl.BlockSpec((1, _H), lambda i: (0, 0)),
            pl.BlockSpec((_H, 2), lambda i: (0, 0)),
            pl.BlockSpec((1, 2), lambda i: (0, 0)),
        ],
        out_specs=pl.BlockSpec((1, 2), lambda i: (0, 0)),
        out_shape=jax.ShapeDtypeStruct((1, 2), jnp.float32),
    )(acc, h2p, dis, b2, lin_W, lin_b)


def kernel(x, edge_index, batch, W1, b1, W2, b2, lin_W, lin_b):
    src2d = edge_index[0].reshape(_EROWS, _LANES)
    dst2d = edge_index[1].reshape(_EROWS, _LANES)

    deg2 = _sc_degree(dst2d)
    h1 = _tca(x, W1)
    deg_col = deg2[:, :_N].reshape(2, _N, 1)
    h1p, dis = _tcb(h1, deg_col)
    acc1 = _sc_scatter(h1p, src2d, dst2d)
    h2p = _tc2(acc1, h1p, dis, b1.reshape(1, _H), W2)
    acc2 = _sc_scatter(h2p, src2d, dst2d)
    return _tc3(acc2, h2p, dis, b2.reshape(1, _H), lin_W, lin_b.reshape(1, 2))


# speedup vs baseline: 49.9442x; 1.0995x over previous
import functools
import jax
import jax.numpy as jnp
from jax import lax
from jax.experimental import pallas as pl
from jax.experimental.pallas import tpu as pltpu
from jax.experimental.pallas import tpu_sc as plsc

_N = 10000
_NACC = 10240
_DIN = 128
_H = 64
_NT = 32
_LANES = 128
_EROWS = 2500
_STEPS = 78
_RB = 2000
_GRID = _N // _RB

_mesh = plsc.VectorSubcoreMesh(core_axis_name="c", subcore_axis_name="s")


@functools.partial(
    pl.kernel,
    mesh=_mesh,
    out_type=jax.ShapeDtypeStruct((2, _NACC), jnp.float32),
    scratch_types=[
        pltpu.VMEM((_STEPS + 1, _LANES), jnp.int32),
        pltpu.VMEM((_NACC // 16,), jnp.float32),
        pltpu.VMEM((_LANES,), jnp.float32),
        pltpu.VMEM_SHARED((_NACC,), jnp.float32),
        pltpu.SemaphoreType.DMA,
    ],
    compiler_params=pltpu.CompilerParams(use_tc_tiling_on_sc=False),
)
def _sc_degree(e_hbm, out_hbm, idx_d, zbuf, obuf, dsh, dsem):
    cid = lax.axis_index("c")
    sid = lax.axis_index("s")
    wid = cid * 16 + sid
    stripe = _NACC // 16

    def zero_body(i, _):
        zbuf[pl.ds(i * 16, 16)] = jnp.zeros((16,), jnp.float32)
        return 0
    lax.fori_loop(0, stripe // 16, zero_body, 0)

    def ones_body(i, _):
        obuf[pl.ds(i * 16, 16)] = jnp.ones((16,), jnp.float32)
        return 0
    lax.fori_loop(0, _LANES // 16, ones_body, 0)

    pltpu.sync_copy(zbuf, dsh.at[pl.ds(sid * stripe, stripe)])
    plsc.subcore_barrier()

    pltpu.sync_copy(e_hbm.at[1, pl.ds(wid * _STEPS, _STEPS)],
                    idx_d.at[pl.ds(0, _STEPS)])

    @pl.when(wid < _EROWS - _NT * _STEPS)
    def _():
        pltpu.sync_copy(e_hbm.at[1, _NT * _STEPS + wid], idx_d.at[_STEPS])

    def acc_body(j, _):
        pltpu.async_copy(obuf, dsh.at[idx_d.at[j]], dsem, add=True)
        return 0
    lax.fori_loop(0, _STEPS, acc_body, 0)

    @pl.when(wid < _EROWS - _NT * _STEPS)
    def _():
        pltpu.async_copy(obuf, dsh.at[idx_d.at[_STEPS]], dsem, add=True)

    def drain_body(j, _):
        pltpu.make_async_copy(obuf, dsh.at[idx_d.at[j]], dsem).wait()
        return 0
    lax.fori_loop(0, _STEPS, drain_body, 0)

    @pl.when(wid < _EROWS - _NT * _STEPS)
    def _():
        pltpu.make_async_copy(obuf, dsh.at[idx_d.at[_STEPS]], dsem).wait()

    plsc.subcore_barrier()
    pltpu.sync_copy(dsh.at[pl.ds(sid * stripe, stripe)],
                    out_hbm.at[cid, pl.ds(sid * stripe, stripe)])


@functools.partial(
    pl.kernel,
    mesh=_mesh,
    out_type=jax.ShapeDtypeStruct((2, _N, _H), jnp.float32),
    scratch_types=[
        pltpu.VMEM((_STEPS + 1, _LANES), jnp.int32),
        pltpu.VMEM((_STEPS + 1, _LANES), jnp.int32),
        pltpu.VMEM((4, _LANES, _H), jnp.float32),
        pltpu.VMEM_SHARED((_NACC, _H), jnp.float32),
        [pltpu.SemaphoreType.DMA] * 4,
        [pltpu.SemaphoreType.DMA] * 4,
    ],
    compiler_params=pltpu.CompilerParams(use_tc_tiling_on_sc=False),
)
def _sc_scatter(h_hbm, e_hbm, out_hbm,
                idx_s, idx_d, rows, acc_sh, gsem, ssem):
    cid = lax.axis_index("c")
    sid = lax.axis_index("s")
    wid = cid * 16 + sid
    zstripe = _NACC // 16
    ostripe = _N // 16
    has_tail = wid < _EROWS - _NT * _STEPS

    def zero_body(i, _):
        r = i // (_H // 16)
        c = (i % (_H // 16)) * 16
        rows[0, r, pl.ds(c, 16)] = jnp.zeros((16,), jnp.float32)
        return 0
    lax.fori_loop(0, _LANES * (_H // 16), zero_body, 0)

    def zcopy_body(i, _):
        pltpu.sync_copy(rows.at[0],
                        acc_sh.at[pl.ds(sid * zstripe + i * _LANES, _LANES)])
        return 0
    lax.fori_loop(0, zstripe // _LANES, zcopy_body, 0)
    plsc.subcore_barrier()

    pltpu.sync_copy(e_hbm.at[0, pl.ds(wid * _STEPS, _STEPS)],
                    idx_s.at[pl.ds(0, _STEPS)])
    pltpu.sync_copy(e_hbm.at[1, pl.ds(wid * _STEPS, _STEPS)],
                    idx_d.at[pl.ds(0, _STEPS)])

    @pl.when(has_tail)
    def _():
        pltpu.sync_copy(e_hbm.at[0, _NT * _STEPS + wid], idx_s.at[_STEPS])
        pltpu.sync_copy(e_hbm.at[1, _NT * _STEPS + wid], idx_d.at[_STEPS])

    def outer(o, _):
        for i in range(4):
            j = o * 4 + i

            @pl.when(j < _STEPS)
            def _():
                @pl.when(j >= 4)
                def _():
                    pltpu.make_async_copy(
                        rows.at[i], acc_sh.at[idx_d.at[j - 4]],
                        ssem[i]).wait()
                pltpu.async_copy(h_hbm.at[idx_s.at[j]], rows.at[i], gsem[i])

            jj = j - 2
            bs = (i + 2) % 4

            @pl.when((0 <= jj) & (jj < _STEPS))
            def _():
                pltpu.make_async_copy(h_hbm.at[idx_s.at[jj]], rows.at[bs],
                                      gsem[bs]).wait()
                pltpu.async_copy(rows.at[bs], acc_sh.at[idx_d.at[jj]],
                                 ssem[bs], add=True)
        return 0
    lax.fori_loop(0, _STEPS // 4 + 1, outer, 0)

    for b in range(4):
        pltpu.make_async_copy(rows.at[b],
                              acc_sh.at[idx_d.at[_STEPS - 4 + (b + 2) % 4]],
                              ssem[b]).wait()

    @pl.when(has_tail)
    def _():
        pltpu.async_copy(h_hbm.at[idx_s.at[_STEPS]], rows.at[0],
                         gsem[0]).wait()
        pltpu.sync_copy(rows.at[0], acc_sh.at[idx_d.at[_STEPS]], add=True)

    plsc.subcore_barrier()
    pltpu.sync_copy(acc_sh.at[pl.ds(sid * ostripe, ostripe)],
                    out_hbm.at[cid, pl.ds(sid * ostripe, ostripe)])


def _tca_body(x_ref, w_ref, h_ref):
    h_ref[...] = jnp.dot(x_ref[...], w_ref[...],
                         preferred_element_type=jnp.float32)


def _tca(x, W1):
    return pl.pallas_call(
        _tca_body,
        grid=(_GRID,),
        in_specs=[
            pl.BlockSpec((_RB, _DIN), lambda i: (i, 0)),
            pl.BlockSpec((_DIN, _H), lambda i: (0, 0)),
        ],
        out_specs=pl.BlockSpec((_RB, _H), lambda i: (i, 0)),
        out_shape=jax.ShapeDtypeStruct((_N, _H), jnp.float32),
    )(x, W1)


def _tcb_body(h_ref, deg_ref, hp_ref):
    deg = deg_ref[0, 0] + deg_ref[0, 1] + 1.0
    dis = lax.rsqrt(deg).reshape(_RB, 1)
    hp_ref[...] = dis * h_ref[...]


def _tcb(h1, deg_col):
    return pl.pallas_call(
        _tcb_body,
        grid=(_GRID,),
        in_specs=[
            pl.BlockSpec((_RB, _H), lambda i: (i, 0)),
            pl.BlockSpec((1, 2, _RB), lambda i: (i, 0, 0)),
        ],
        out_specs=pl.BlockSpec((_RB, _H), lambda i: (i, 0)),
        out_shape=jax.ShapeDtypeStruct((_N, _H), jnp.float32),
    )(h1, deg_col)


def _tc2_body(acc_ref, hp_ref, deg_ref, b_ref, w_ref, out_ref):
    acc = acc_ref[0] + acc_ref[1] + hp_ref[...]
    deg = deg_ref[0, 0] + deg_ref[0, 1] + 1.0
    dis = lax.rsqrt(deg).reshape(_RB, 1)
    z = jnp.maximum(dis * acc + b_ref[...], 0.0)
    h2 = jnp.dot(z, w_ref[...], preferred_element_type=jnp.float32)
    out_ref[...] = dis * h2


def _tc2(acc, h1p, deg3, b1, W2):
    return pl.pallas_call(
        _tc2_body,
        grid=(_GRID,),
        in_specs=[
            pl.BlockSpec((2, _RB, _H), lambda i: (0, i, 0)),
            pl.BlockSpec((_RB, _H), lambda i: (i, 0)),
            pl.BlockSpec((1, 2, _RB), lambda i: (i, 0, 0)),
            pl.BlockSpec((1, _H), lambda i: (0, 0)),
            pl.BlockSpec((_H, _H), lambda i: (0, 0)),
        ],
        out_specs=pl.BlockSpec((_RB, _H), lambda i: (i, 0)),
        out_shape=jax.ShapeDtypeStruct((_N, _H), jnp.float32),
    )(acc, h1p, deg3, b1, W2)


def _tc3_body(acc_ref, hp_ref, deg_ref, b_ref, lw_ref, lb_ref, out_ref):
    i = pl.program_id(0)
    acc = acc_ref[0] + acc_ref[1] + hp_ref[...]
    deg = deg_ref[0, 0] + deg_ref[0, 1] + 1.0
    dis = lax.rsqrt(deg).reshape(_RB, 1)
    z = jnp.maximum(dis * acc + b_ref[...], 0.0)
    part = jnp.sum(z, axis=0, keepdims=True) * (1.0 / _N)
    contrib = jnp.dot(part, lw_ref[...], preferred_element_type=jnp.float32)

    @pl.when(i == 0)
    def _():
        out_ref[...] = lb_ref[...]
    out_ref[...] += contrib


def _tc3(acc, h2p, deg3, b2, lin_W, lin_b):
    return pl.pallas_call(
        _tc3_body,
        grid=(_GRID,),
        in_specs=[
            pl.BlockSpec((2, _RB, _H), lambda i: (0, i, 0)),
            pl.BlockSpec((_RB, _H), lambda i: (i, 0)),
            pl.BlockSpec((1, 2, _RB), lambda i: (i, 0, 0)),
            pl.BlockSpec((1, _H), lambda i: (0, 0)),
            pl.BlockSpec((_H, 2), lambda i: (0, 0)),
            pl.BlockSpec((1, 2), lambda i: (0, 0)),
        ],
        out_specs=pl.BlockSpec((1, 2), lambda i: (0, 0)),
        out_shape=jax.ShapeDtypeStruct((1, 2), jnp.float32),
    )(acc, h2p, deg3, b2, lin_W, lin_b)


def kernel(x, edge_index, batch, W1, b1, W2, b2, lin_W, lin_b):
    e3 = edge_index.reshape(2, _EROWS, _LANES)

    deg2 = _sc_degree(e3)
    h1 = _tca(x, W1)
    deg3 = deg2[:, :_N].reshape(2, _GRID, _RB).transpose(1, 0, 2)
    h1p = _tcb(h1, deg3)
    acc1 = _sc_scatter(h1p, e3)
    h2p = _tc2(acc1, h1p, deg3, b1.reshape(1, _H), W2)
    acc2 = _sc_scatter(h2p, e3)
    return _tc3(acc2, h2p, deg3, b2.reshape(1, _H), lin_W, lin_b.reshape(1, 2))


# speedup vs baseline: 54.1428x; 1.0841x over previous
import functools
import jax
import jax.numpy as jnp
from jax import lax
from jax.experimental import pallas as pl
from jax.experimental.pallas import tpu as pltpu
from jax.experimental.pallas import tpu_sc as plsc

_N = 10000
_NACC = 10240
_DIN = 128
_H = 64
_NT = 32
_LANES = 128
_EROWS = 2500
_STEPS = 78
_RB = 5000
_GRID = _N // _RB

_mesh = plsc.VectorSubcoreMesh(core_axis_name="c", subcore_axis_name="s")


@functools.partial(
    pl.kernel,
    mesh=_mesh,
    out_type=jax.ShapeDtypeStruct((2, _NACC), jnp.float32),
    scratch_types=[
        pltpu.VMEM((_STEPS + 1, _LANES), jnp.int32),
        pltpu.VMEM((_NACC // 16,), jnp.float32),
        pltpu.VMEM((_LANES,), jnp.float32),
        pltpu.VMEM_SHARED((_NACC,), jnp.float32),
        pltpu.SemaphoreType.DMA,
    ],
    compiler_params=pltpu.CompilerParams(use_tc_tiling_on_sc=False),
)
def _sc_degree(e_hbm, out_hbm, idx_d, zbuf, obuf, dsh, dsem):
    cid = lax.axis_index("c")
    sid = lax.axis_index("s")
    wid = cid * 16 + sid
    stripe = _NACC // 16

    def zero_body(i, _):
        zbuf[pl.ds(i * 16, 16)] = jnp.zeros((16,), jnp.float32)
        return 0
    lax.fori_loop(0, stripe // 16, zero_body, 0)

    def ones_body(i, _):
        obuf[pl.ds(i * 16, 16)] = jnp.ones((16,), jnp.float32)
        return 0
    lax.fori_loop(0, _LANES // 16, ones_body, 0)

    pltpu.sync_copy(zbuf, dsh.at[pl.ds(sid * stripe, stripe)])
    plsc.subcore_barrier()

    pltpu.sync_copy(e_hbm.at[1, pl.ds(wid * _STEPS, _STEPS)],
                    idx_d.at[pl.ds(0, _STEPS)])

    @pl.when(wid < _EROWS - _NT * _STEPS)
    def _():
        pltpu.sync_copy(e_hbm.at[1, _NT * _STEPS + wid], idx_d.at[_STEPS])

    def acc_body(j, _):
        pltpu.async_copy(obuf, dsh.at[idx_d.at[j]], dsem, add=True)
        return 0
    lax.fori_loop(0, _STEPS, acc_body, 0)

    @pl.when(wid < _EROWS - _NT * _STEPS)
    def _():
        pltpu.async_copy(obuf, dsh.at[idx_d.at[_STEPS]], dsem, add=True)

    def drain_body(j, _):
        pltpu.make_async_copy(obuf, dsh.at[idx_d.at[j]], dsem).wait()
        return 0
    lax.fori_loop(0, _STEPS, drain_body, 0)

    @pl.when(wid < _EROWS - _NT * _STEPS)
    def _():
        pltpu.make_async_copy(obuf, dsh.at[idx_d.at[_STEPS]], dsem).wait()

    plsc.subcore_barrier()
    pltpu.sync_copy(dsh.at[pl.ds(sid * stripe, stripe)],
                    out_hbm.at[cid, pl.ds(sid * stripe, stripe)])


@functools.partial(
    pl.kernel,
    mesh=_mesh,
    out_type=jax.ShapeDtypeStruct((2, _N, _H), jnp.float32),
    scratch_types=[
        pltpu.VMEM((_STEPS + 1, _LANES), jnp.int32),
        pltpu.VMEM((_STEPS + 1, _LANES), jnp.int32),
        pltpu.VMEM((6, _LANES, _H), jnp.float32),
        pltpu.VMEM_SHARED((_NACC, _H), jnp.float32),
        [pltpu.SemaphoreType.DMA] * 6,
        [pltpu.SemaphoreType.DMA] * 6,
    ],
    compiler_params=pltpu.CompilerParams(use_tc_tiling_on_sc=False),
)
def _sc_scatter(h_hbm, e_hbm, out_hbm,
                idx_s, idx_d, rows, acc_sh, gsem, ssem):
    cid = lax.axis_index("c")
    sid = lax.axis_index("s")
    wid = cid * 16 + sid
    zstripe = _NACC // 16
    ostripe = _N // 16
    has_tail = wid < _EROWS - _NT * _STEPS

    def zero_body(i, _):
        r = i // (_H // 16)
        c = (i % (_H // 16)) * 16
        rows[0, r, pl.ds(c, 16)] = jnp.zeros((16,), jnp.float32)
        return 0
    lax.fori_loop(0, _LANES * (_H // 16), zero_body, 0)

    def zcopy_body(i, _):
        pltpu.sync_copy(rows.at[0],
                        acc_sh.at[pl.ds(sid * zstripe + i * _LANES, _LANES)])
        return 0
    lax.fori_loop(0, zstripe // _LANES, zcopy_body, 0)
    plsc.subcore_barrier()

    pltpu.sync_copy(e_hbm.at[0, pl.ds(wid * _STEPS, _STEPS)],
                    idx_s.at[pl.ds(0, _STEPS)])
    pltpu.sync_copy(e_hbm.at[1, pl.ds(wid * _STEPS, _STEPS)],
                    idx_d.at[pl.ds(0, _STEPS)])

    @pl.when(has_tail)
    def _():
        pltpu.sync_copy(e_hbm.at[0, _NT * _STEPS + wid], idx_s.at[_STEPS])
        pltpu.sync_copy(e_hbm.at[1, _NT * _STEPS + wid], idx_d.at[_STEPS])

    def outer(o, _):
        for i in range(6):
            j = o * 6 + i

            @pl.when(j < _STEPS)
            def _():
                @pl.when(j >= 6)
                def _():
                    pltpu.make_async_copy(
                        rows.at[i], acc_sh.at[idx_d.at[j - 6]],
                        ssem[i]).wait()
                pltpu.async_copy(h_hbm.at[idx_s.at[j]], rows.at[i], gsem[i])

            jj = j - 3
            bs = (i + 3) % 6

            @pl.when((0 <= jj) & (jj < _STEPS))
            def _():
                pltpu.make_async_copy(h_hbm.at[idx_s.at[jj]], rows.at[bs],
                                      gsem[bs]).wait()
                pltpu.async_copy(rows.at[bs], acc_sh.at[idx_d.at[jj]],
                                 ssem[bs], add=True)
        return 0
    lax.fori_loop(0, _STEPS // 6 + 1, outer, 0)

    for b in range(6):
        pltpu.make_async_copy(rows.at[b],
                              acc_sh.at[idx_d.at[_STEPS - 6 + b]],
                              ssem[b]).wait()

    @pl.when(has_tail)
    def _():
        pltpu.async_copy(h_hbm.at[idx_s.at[_STEPS]], rows.at[0],
                         gsem[0]).wait()
        pltpu.sync_copy(rows.at[0], acc_sh.at[idx_d.at[_STEPS]], add=True)

    plsc.subcore_barrier()
    pltpu.sync_copy(acc_sh.at[pl.ds(sid * ostripe, ostripe)],
                    out_hbm.at[cid, pl.ds(sid * ostripe, ostripe)])


def _tca_body(x_ref, w_ref, h_ref):
    h_ref[...] = jnp.dot(x_ref[...], w_ref[...],
                         preferred_element_type=jnp.float32)


def _tca(x, W1):
    return pl.pallas_call(
        _tca_body,
        grid=(_GRID,),
        in_specs=[
            pl.BlockSpec((_RB, _DIN), lambda i: (i, 0)),
            pl.BlockSpec((_DIN, _H), lambda i: (0, 0)),
        ],
        out_specs=pl.BlockSpec((_RB, _H), lambda i: (i, 0)),
        out_shape=jax.ShapeDtypeStruct((_N, _H), jnp.float32),
    )(x, W1)


def _tcb_body(h_ref, deg_ref, hp_ref):
    deg = deg_ref[0, 0] + deg_ref[0, 1] + 1.0
    dis = lax.rsqrt(deg).reshape(_RB, 1)
    hp_ref[...] = dis * h_ref[...]


def _tcb(h1, deg_col):
    return pl.pallas_call(
        _tcb_body,
        grid=(_GRID,),
        in_specs=[
            pl.BlockSpec((_RB, _H), lambda i: (i, 0)),
            pl.BlockSpec((1, 2, _RB), lambda i: (i, 0, 0)),
        ],
        out_specs=pl.BlockSpec((_RB, _H), lambda i: (i, 0)),
        out_shape=jax.ShapeDtypeStruct((_N, _H), jnp.float32),
    )(h1, deg_col)


def _tc2_body(acc_ref, hp_ref, deg_ref, b_ref, w_ref, out_ref):
    acc = acc_ref[0] + acc_ref[1] + hp_ref[...]
    deg = deg_ref[0, 0] + deg_ref[0, 1] + 1.0
    dis = lax.rsqrt(deg).reshape(_RB, 1)
    z = jnp.maximum(dis * acc + b_ref[...], 0.0)
    h2 = jnp.dot(z, w_ref[...], preferred_element_type=jnp.float32)
    out_ref[...] = dis * h2


def _tc2(acc, h1p, deg3, b1, W2):
    return pl.pallas_call(
        _tc2_body,
        grid=(_GRID,),
        in_specs=[
            pl.BlockSpec((2, _RB, _H), lambda i: (0, i, 0)),
            pl.BlockSpec((_RB, _H), lambda i: (i, 0)),
            pl.BlockSpec((1, 2, _RB), lambda i: (i, 0, 0)),
            pl.BlockSpec((1, _H), lambda i: (0, 0)),
            pl.BlockSpec((_H, _H), lambda i: (0, 0)),
        ],
        out_specs=pl.BlockSpec((_RB, _H), lambda i: (i, 0)),
        out_shape=jax.ShapeDtypeStruct((_N, _H), jnp.float32),
    )(acc, h1p, deg3, b1, W2)


def _tc3_body(acc_ref, hp_ref, deg_ref, b_ref, lw_ref, lb_ref, out_ref):
    i = pl.program_id(0)
    acc = acc_ref[0] + acc_ref[1] + hp_ref[...]
    deg = deg_ref[0, 0] + deg_ref[0, 1] + 1.0
    dis = lax.rsqrt(deg).reshape(_RB, 1)
    z = jnp.maximum(dis * acc + b_ref[...], 0.0)
    part = jnp.sum(z, axis=0, keepdims=True) * (1.0 / _N)
    contrib = jnp.dot(part, lw_ref[...], preferred_element_type=jnp.float32)

    @pl.when(i == 0)
    def _():
        out_ref[...] = lb_ref[...]
    out_ref[...] += contrib


def _tc3(acc, h2p, deg3, b2, lin_W, lin_b):
    return pl.pallas_call(
        _tc3_body,
        grid=(_GRID,),
        in_specs=[
            pl.BlockSpec((2, _RB, _H), lambda i: (0, i, 0)),
            pl.BlockSpec((_RB, _H), lambda i: (i, 0)),
            pl.BlockSpec((1, 2, _RB), lambda i: (i, 0, 0)),
            pl.BlockSpec((1, _H), lambda i: (0, 0)),
            pl.BlockSpec((_H, 2), lambda i: (0, 0)),
            pl.BlockSpec((1, 2), lambda i: (0, 0)),
        ],
        out_specs=pl.BlockSpec((1, 2), lambda i: (0, 0)),
        out_shape=jax.ShapeDtypeStruct((1, 2), jnp.float32),
    )(acc, h2p, deg3, b2, lin_W, lin_b)


def kernel(x, edge_index, batch, W1, b1, W2, b2, lin_W, lin_b):
    e3 = edge_index.reshape(2, _EROWS, _LANES)

    deg2 = _sc_degree(e3)
    h1 = _tca(x, W1)
    deg3 = deg2[:, :_N].reshape(2, _GRID, _RB).transpose(1, 0, 2)
    h1p = _tcb(h1, deg3)
    acc1 = _sc_scatter(h1p, e3)
    h2p = _tc2(acc1, h1p, deg3, b1.reshape(1, _H), W2)
    acc2 = _sc_scatter(h2p, e3)
    return _tc3(acc2, h2p, deg3, b2.reshape(1, _H), lin_W, lin_b.reshape(1, 2))


# speedup vs baseline: 60.1817x; 1.1115x over previous
import functools
import jax
import jax.numpy as jnp
from jax import lax
from jax.experimental import pallas as pl
from jax.experimental.pallas import tpu as pltpu
from jax.experimental.pallas import tpu_sc as plsc

_N = 10000
_NACC = 10240
_DIN = 128
_H = 64
_NT = 32
_LANES = 128
_EROWS = 2500
_STEPS = 78
_RB = 5000
_GRID = _N // _RB
_RBP = 1000
_GRIDP = (_N // 2) // _RBP

_mesh = plsc.VectorSubcoreMesh(core_axis_name="c", subcore_axis_name="s")


@functools.partial(
    pl.kernel,
    mesh=_mesh,
    out_type=jax.ShapeDtypeStruct((2, _NACC), jnp.float32),
    scratch_types=[
        pltpu.VMEM((_STEPS + 1, _LANES), jnp.int32),
        pltpu.VMEM((_NACC // 16,), jnp.float32),
        pltpu.VMEM((_LANES,), jnp.float32),
        pltpu.VMEM_SHARED((_NACC,), jnp.float32),
        pltpu.SemaphoreType.DMA,
    ],
    compiler_params=pltpu.CompilerParams(use_tc_tiling_on_sc=False),
)
def _sc_degree(e_hbm, out_hbm, idx_d, zbuf, obuf, dsh, dsem):
    cid = lax.axis_index("c")
    sid = lax.axis_index("s")
    wid = cid * 16 + sid
    stripe = _NACC // 16

    def zero_body(i, _):
        zbuf[pl.ds(i * 16, 16)] = jnp.zeros((16,), jnp.float32)
        return 0
    lax.fori_loop(0, stripe // 16, zero_body, 0)

    def ones_body(i, _):
        obuf[pl.ds(i * 16, 16)] = jnp.ones((16,), jnp.float32)
        return 0
    lax.fori_loop(0, _LANES // 16, ones_body, 0)

    pltpu.sync_copy(zbuf, dsh.at[pl.ds(sid * stripe, stripe)])
    plsc.subcore_barrier()

    pltpu.sync_copy(e_hbm.at[1, pl.ds(wid * _STEPS, _STEPS)],
                    idx_d.at[pl.ds(0, _STEPS)])

    @pl.when(wid < _EROWS - _NT * _STEPS)
    def _():
        pltpu.sync_copy(e_hbm.at[1, _NT * _STEPS + wid], idx_d.at[_STEPS])

    def acc_body(j, _):
        pltpu.async_copy(obuf, dsh.at[idx_d.at[j]], dsem, add=True)
        return 0
    lax.fori_loop(0, _STEPS, acc_body, 0)

    @pl.when(wid < _EROWS - _NT * _STEPS)
    def _():
        pltpu.async_copy(obuf, dsh.at[idx_d.at[_STEPS]], dsem, add=True)

    def drain_body(j, _):
        pltpu.make_async_copy(obuf, dsh.at[idx_d.at[j]], dsem).wait()
        return 0
    lax.fori_loop(0, _STEPS, drain_body, 0)

    @pl.when(wid < _EROWS - _NT * _STEPS)
    def _():
        pltpu.make_async_copy(obuf, dsh.at[idx_d.at[_STEPS]], dsem).wait()

    plsc.subcore_barrier()
    pltpu.sync_copy(dsh.at[pl.ds(sid * stripe, stripe)],
                    out_hbm.at[cid, pl.ds(sid * stripe, stripe)])


@functools.partial(
    pl.kernel,
    mesh=_mesh,
    out_type=jax.ShapeDtypeStruct((2, _N, _H), jnp.float32),
    scratch_types=[
        pltpu.VMEM((_STEPS + 1, _LANES), jnp.int32),
        pltpu.VMEM((_STEPS + 1, _LANES), jnp.int32),
        pltpu.VMEM((6, _LANES, _H), jnp.float32),
        pltpu.VMEM_SHARED((_NACC, _H), jnp.float32),
        [pltpu.SemaphoreType.DMA] * 6,
        [pltpu.SemaphoreType.DMA] * 6,
    ],
    compiler_params=pltpu.CompilerParams(use_tc_tiling_on_sc=False),
)
def _sc_scatter(h_hbm, e_hbm, out_hbm,
                idx_s, idx_d, rows, acc_sh, gsem, ssem):
    cid = lax.axis_index("c")
    sid = lax.axis_index("s")
    wid = cid * 16 + sid
    zstripe = _NACC // 16
    ostripe = _N // 16
    has_tail = wid < _EROWS - _NT * _STEPS

    def zero_body(i, _):
        r = i // (_H // 16)
        c = (i % (_H // 16)) * 16
        rows[0, r, pl.ds(c, 16)] = jnp.zeros((16,), jnp.float32)
        return 0
    lax.fori_loop(0, _LANES * (_H // 16), zero_body, 0)

    def zcopy_body(i, _):
        pltpu.sync_copy(rows.at[0],
                        acc_sh.at[pl.ds(sid * zstripe + i * _LANES, _LANES)])
        return 0
    lax.fori_loop(0, zstripe // _LANES, zcopy_body, 0)
    plsc.subcore_barrier()

    pltpu.sync_copy(e_hbm.at[0, pl.ds(wid * _STEPS, _STEPS)],
                    idx_s.at[pl.ds(0, _STEPS)])
    pltpu.sync_copy(e_hbm.at[1, pl.ds(wid * _STEPS, _STEPS)],
                    idx_d.at[pl.ds(0, _STEPS)])

    @pl.when(has_tail)
    def _():
        pltpu.sync_copy(e_hbm.at[0, _NT * _STEPS + wid], idx_s.at[_STEPS])
        pltpu.sync_copy(e_hbm.at[1, _NT * _STEPS + wid], idx_d.at[_STEPS])

    def outer(o, _):
        for i in range(6):
            j = o * 6 + i

            @pl.when(j < _STEPS)
            def _():
                @pl.when(j >= 6)
                def _():
                    pltpu.make_async_copy(
                        rows.at[i], acc_sh.at[idx_d.at[j - 6]],
                        ssem[i]).wait()
                pltpu.async_copy(h_hbm.at[idx_s.at[j]], rows.at[i], gsem[i])

            jj = j - 3
            bs = (i + 3) % 6

            @pl.when((0 <= jj) & (jj < _STEPS))
            def _():
                pltpu.make_async_copy(h_hbm.at[idx_s.at[jj]], rows.at[bs],
                                      gsem[bs]).wait()
                pltpu.async_copy(rows.at[bs], acc_sh.at[idx_d.at[jj]],
                                 ssem[bs], add=True)
        return 0
    lax.fori_loop(0, _STEPS // 6 + 1, outer, 0)

    for b in range(6):
        pltpu.make_async_copy(rows.at[b],
                              acc_sh.at[idx_d.at[_STEPS - 6 + b]],
                              ssem[b]).wait()

    @pl.when(has_tail)
    def _():
        pltpu.async_copy(h_hbm.at[idx_s.at[_STEPS]], rows.at[0],
                         gsem[0]).wait()
        pltpu.sync_copy(rows.at[0], acc_sh.at[idx_d.at[_STEPS]], add=True)

    plsc.subcore_barrier()
    pltpu.sync_copy(acc_sh.at[pl.ds(sid * ostripe, ostripe)],
                    out_hbm.at[cid, pl.ds(sid * ostripe, ostripe)])


def _tca_body(x_ref, w_ref, h_ref):
    h_ref[...] = jnp.dot(x_ref[...], w_ref[...],
                         preferred_element_type=jnp.float32)


def _tca(x, W1):
    return pl.pallas_call(
        _tca_body,
        grid=(_GRID,),
        in_specs=[
            pl.BlockSpec((_RB, _DIN), lambda i: (i, 0)),
            pl.BlockSpec((_DIN, _H), lambda i: (0, 0)),
        ],
        out_specs=pl.BlockSpec((_RB, _H), lambda i: (i, 0)),
        out_shape=jax.ShapeDtypeStruct((_N, _H), jnp.float32),
    )(x, W1)


def _tcb_body(ht_ref, hb_ref, deg_ref, hp_ref):
    deg = deg_ref[0, 0] + deg_ref[1, 0] + 1.0
    dist = lax.rsqrt(deg[:, 0:1])
    disb = lax.rsqrt(deg[:, 1:2])
    hp_ref[:, 0:_H] = dist * ht_ref[...]
    hp_ref[:, _H:2 * _H] = disb * hb_ref[...]


def _tcb(h1, deg_p):
    return pl.pallas_call(
        _tcb_body,
        grid=(_GRIDP,),
        in_specs=[
            pl.BlockSpec((_RBP, _H), lambda i: (i, 0)),
            pl.BlockSpec((_RBP, _H), lambda i: (i + _GRIDP, 0)),
            pl.BlockSpec((2, 1, _RBP, 2), lambda i: (0, i, 0, 0)),
        ],
        out_specs=pl.BlockSpec((_RBP, 2 * _H), lambda i: (i, 0)),
        out_shape=jax.ShapeDtypeStruct((_N // 2, 2 * _H), jnp.float32),
    )(h1, h1, deg_p)


def _dis128(deg_ref):
    deg = deg_ref[0, 0] + deg_ref[1, 0] + 1.0
    dist = jnp.broadcast_to(lax.rsqrt(deg[:, 0:1]), (_RBP, _H))
    disb = jnp.broadcast_to(lax.rsqrt(deg[:, 1:2]), (_RBP, _H))
    return jnp.concatenate([dist, disb], axis=1)


def _tc2_body(acc_ref, hp_ref, deg_ref, b_ref, w_ref, out_ref):
    acc = acc_ref[0] + acc_ref[1] + hp_ref[...]
    dis = _dis128(deg_ref)
    z = jnp.maximum(dis * acc + b_ref[...], 0.0)
    h2 = jnp.dot(z, w_ref[...], preferred_element_type=jnp.float32)
    out_ref[...] = dis * h2


def _tc2(acc, h1p, deg_p, b1, W2):
    return pl.pallas_call(
        _tc2_body,
        grid=(_GRIDP,),
        in_specs=[
            pl.BlockSpec((2, _RBP, 2 * _H), lambda i: (0, i, 0)),
            pl.BlockSpec((_RBP, 2 * _H), lambda i: (i, 0)),
            pl.BlockSpec((2, 1, _RBP, 2), lambda i: (0, i, 0, 0)),
            pl.BlockSpec((1, 2 * _H), lambda i: (0, 0)),
            pl.BlockSpec((2 * _H, 2 * _H), lambda i: (0, 0)),
        ],
        out_specs=pl.BlockSpec((_RBP, 2 * _H), lambda i: (i, 0)),
        out_shape=jax.ShapeDtypeStruct((_N // 2, 2 * _H), jnp.float32),
    )(acc, h1p, deg_p, b1, W2)


def _tc3_body(acc_ref, hp_ref, deg_ref, b_ref, lw_ref, lb_ref, out_ref):
    i = pl.program_id(0)
    acc = acc_ref[0] + acc_ref[1] + hp_ref[...]
    dis = _dis128(deg_ref)
    z = jnp.maximum(dis * acc + b_ref[...], 0.0)
    part = jnp.sum(z, axis=0, keepdims=True) * (1.0 / _N)
    pooled = part[:, 0:_H] + part[:, _H:2 * _H]
    contrib = jnp.dot(pooled, lw_ref[...], preferred_element_type=jnp.float32)

    @pl.when(i == 0)
    def _():
        out_ref[...] = lb_ref[...]
    out_ref[...] += contrib


def _tc3(acc, h2p, deg_p, b2, lin_W, lin_b):
    return pl.pallas_call(
        _tc3_body,
        grid=(_GRIDP,),
        in_specs=[
            pl.BlockSpec((2, _RBP, 2 * _H), lambda i: (0, i, 0)),
            pl.BlockSpec((_RBP, 2 * _H), lambda i: (i, 0)),
            pl.BlockSpec((2, 1, _RBP, 2), lambda i: (0, i, 0, 0)),
            pl.BlockSpec((1, 2 * _H), lambda i: (0, 0)),
            pl.BlockSpec((_H, 2), lambda i: (0, 0)),
            pl.BlockSpec((1, 2), lambda i: (0, 0)),
        ],
        out_specs=pl.BlockSpec((1, 2), lambda i: (0, 0)),
        out_shape=jax.ShapeDtypeStruct((1, 2), jnp.float32),
    )(acc, h2p, deg_p, b2, lin_W, lin_b)


def kernel(x, edge_index, batch, W1, b1, W2, b2, lin_W, lin_b):
    half = _N // 2
    pe = jnp.where(edge_index < half, edge_index * 2,
                   (edge_index - half) * 2 + 1)
    e3 = pe.reshape(2, _EROWS, _LANES)

    deg2 = _sc_degree(e3)
    h1 = _tca(x, W1)
    deg_p = deg2[:, :_N].reshape(2, _GRIDP, _RBP, 2)
    h1p = _tcb(h1, deg_p)
    acc1 = _sc_scatter(h1p.reshape(_N, _H), e3)
    acc1p = acc1.reshape(2, _N // 2, 2 * _H)
    b1p = jnp.concatenate([b1, b1]).reshape(1, 2 * _H)
    W2b = jax.scipy.linalg.block_diag(W2, W2)
    h2p = _tc2(acc1p, h1p, deg_p, b1p, W2b)
    acc2 = _sc_scatter(h2p.reshape(_N, _H), e3)
    acc2p = acc2.reshape(2, _N // 2, 2 * _H)
    b2p = jnp.concatenate([b2, b2]).reshape(1, 2 * _H)
    return _tc3(acc2p, h2p, deg_p, b2p, lin_W, lin_b.reshape(1, 2))


# speedup vs baseline: 60.2283x; 1.0008x over previous
import functools
import jax
import jax.numpy as jnp
from jax import lax
from jax.experimental import pallas as pl
from jax.experimental.pallas import tpu as pltpu
from jax.experimental.pallas import tpu_sc as plsc

_N = 10000
_NACC = 10240
_DIN = 128
_H = 64
_NT = 32
_LANES = 128
_EROWS = 2500
_STEPS = 78
_RB = 5000
_GRID = _N // _RB
_RBP = 1000
_GRIDP = (_N // 2) // _RBP

_mesh = plsc.VectorSubcoreMesh(core_axis_name="c", subcore_axis_name="s")


@functools.partial(
    pl.kernel,
    mesh=_mesh,
    out_type=jax.ShapeDtypeStruct((2, _NACC), jnp.float32),
    scratch_types=[
        pltpu.VMEM((_STEPS + 1, _LANES), jnp.int32),
        pltpu.VMEM((_NACC // 16,), jnp.float32),
        pltpu.VMEM((_LANES,), jnp.float32),
        pltpu.VMEM_SHARED((_NACC,), jnp.float32),
        pltpu.SemaphoreType.DMA,
    ],
    compiler_params=pltpu.CompilerParams(use_tc_tiling_on_sc=False),
)
def _sc_degree(e_hbm, out_hbm, idx_d, zbuf, obuf, dsh, dsem):
    cid = lax.axis_index("c")
    sid = lax.axis_index("s")
    wid = cid * 16 + sid
    stripe = _NACC // 16

    def zero_body(i, _):
        zbuf[pl.ds(i * 16, 16)] = jnp.zeros((16,), jnp.float32)
        return 0
    lax.fori_loop(0, stripe // 16, zero_body, 0)

    def ones_body(i, _):
        obuf[pl.ds(i * 16, 16)] = jnp.ones((16,), jnp.float32)
        return 0
    lax.fori_loop(0, _LANES // 16, ones_body, 0)

    pltpu.sync_copy(zbuf, dsh.at[pl.ds(sid * stripe, stripe)])
    plsc.subcore_barrier()

    pltpu.sync_copy(e_hbm.at[1, pl.ds(wid * _STEPS, _STEPS)],
                    idx_d.at[pl.ds(0, _STEPS)])

    @pl.when(wid < _EROWS - _NT * _STEPS)
    def _():
        pltpu.sync_copy(e_hbm.at[1, _NT * _STEPS + wid], idx_d.at[_STEPS])

    def acc_body(j, _):
        pltpu.async_copy(obuf, dsh.at[idx_d.at[j]], dsem, add=True)
        return 0
    lax.fori_loop(0, _STEPS, acc_body, 0)

    @pl.when(wid < _EROWS - _NT * _STEPS)
    def _():
        pltpu.async_copy(obuf, dsh.at[idx_d.at[_STEPS]], dsem, add=True)

    def drain_body(j, _):
        pltpu.make_async_copy(obuf, dsh.at[idx_d.at[j]], dsem).wait()
        return 0
    lax.fori_loop(0, _STEPS, drain_body, 0)

    @pl.when(wid < _EROWS - _NT * _STEPS)
    def _():
        pltpu.make_async_copy(obuf, dsh.at[idx_d.at[_STEPS]], dsem).wait()

    plsc.subcore_barrier()
    pltpu.sync_copy(dsh.at[pl.ds(sid * stripe, stripe)],
                    out_hbm.at[cid, pl.ds(sid * stripe, stripe)])


@functools.partial(
    pl.kernel,
    mesh=_mesh,
    out_type=jax.ShapeDtypeStruct((2, _N, _H), jnp.float32),
    scratch_types=[
        pltpu.VMEM((_STEPS + 1, _LANES), jnp.int32),
        pltpu.VMEM((_STEPS + 1, _LANES), jnp.int32),
        pltpu.VMEM((6, _LANES, _H), jnp.float32),
        pltpu.VMEM_SHARED((_NACC, _H), jnp.float32),
        [pltpu.SemaphoreType.DMA] * 6,
        [pltpu.SemaphoreType.DMA] * 6,
    ],
    compiler_params=pltpu.CompilerParams(use_tc_tiling_on_sc=False),
)
def _sc_scatter(h_hbm, e_hbm, out_hbm,
                idx_s, idx_d, rows, acc_sh, gsem, ssem):
    cid = lax.axis_index("c")
    sid = lax.axis_index("s")
    wid = cid * 16 + sid
    zstripe = _NACC // 16
    ostripe = _N // 16
    has_tail = wid < _EROWS - _NT * _STEPS

    def zero_body(i, _):
        r = i // (_H // 16)
        c = (i % (_H // 16)) * 16
        rows[0, r, pl.ds(c, 16)] = jnp.zeros((16,), jnp.float32)
        return 0
    lax.fori_loop(0, _LANES * (_H // 16), zero_body, 0)

    def zcopy_body(i, _):
        pltpu.sync_copy(rows.at[0],
                        acc_sh.at[pl.ds(sid * zstripe + i * _LANES, _LANES)])
        return 0
    lax.fori_loop(0, zstripe // _LANES, zcopy_body, 0)
    plsc.subcore_barrier()

    pltpu.sync_copy(e_hbm.at[0, pl.ds(wid * _STEPS, _STEPS)],
                    idx_s.at[pl.ds(0, _STEPS)])
    pltpu.sync_copy(e_hbm.at[1, pl.ds(wid * _STEPS, _STEPS)],
                    idx_d.at[pl.ds(0, _STEPS)])

    @pl.when(has_tail)
    def _():
        pltpu.sync_copy(e_hbm.at[0, _NT * _STEPS + wid], idx_s.at[_STEPS])
        pltpu.sync_copy(e_hbm.at[1, _NT * _STEPS + wid], idx_d.at[_STEPS])

    def outer(o, _):
        for i in range(6):
            j = o * 6 + i

            @pl.when(j < _STEPS)
            def _():
                @pl.when(j >= 6)
                def _():
                    pltpu.make_async_copy(
                        rows.at[i], acc_sh.at[idx_d.at[j - 6]],
                        ssem[i]).wait()
                pltpu.async_copy(h_hbm.at[idx_s.at[j]], rows.at[i], gsem[i])

            jj = j - 3
            bs = (i + 3) % 6

            @pl.when((0 <= jj) & (jj < _STEPS))
            def _():
                pltpu.make_async_copy(h_hbm.at[idx_s.at[jj]], rows.at[bs],
                                      gsem[bs]).wait()
                pltpu.async_copy(rows.at[bs], acc_sh.at[idx_d.at[jj]],
                                 ssem[bs], add=True)
        return 0
    lax.fori_loop(0, _STEPS // 6 + 1, outer, 0)

    for b in range(6):
        pltpu.make_async_copy(rows.at[b],
                              acc_sh.at[idx_d.at[_STEPS - 6 + b]],
                              ssem[b]).wait()

    @pl.when(has_tail)
    def _():
        pltpu.async_copy(h_hbm.at[idx_s.at[_STEPS]], rows.at[0],
                         gsem[0]).wait()
        pltpu.sync_copy(rows.at[0], acc_sh.at[idx_d.at[_STEPS]], add=True)

    plsc.subcore_barrier()
    pltpu.sync_copy(acc_sh.at[pl.ds(sid * ostripe, ostripe)],
                    out_hbm.at[cid, pl.ds(sid * ostripe, ostripe)])


def _tca_body(x_ref, w_ref, h_ref):
    h_ref[...] = jnp.dot(x_ref[...].astype(jnp.bfloat16),
                         w_ref[...].astype(jnp.bfloat16),
                         preferred_element_type=jnp.float32)


def _tca(x, W1):
    return pl.pallas_call(
        _tca_body,
        grid=(_GRID,),
        in_specs=[
            pl.BlockSpec((_RB, _DIN), lambda i: (i, 0)),
            pl.BlockSpec((_DIN, _H), lambda i: (0, 0)),
        ],
        out_specs=pl.BlockSpec((_RB, _H), lambda i: (i, 0)),
        out_shape=jax.ShapeDtypeStruct((_N, _H), jnp.float32),
    )(x, W1)


def _tcb_body(ht_ref, hb_ref, deg_ref, hp_ref):
    deg = deg_ref[0, 0] + deg_ref[1, 0] + 1.0
    dist = lax.rsqrt(deg[:, 0:1])
    disb = lax.rsqrt(deg[:, 1:2])
    hp_ref[:, 0:_H] = dist * ht_ref[...]
    hp_ref[:, _H:2 * _H] = disb * hb_ref[...]


def _tcb(h1, deg_p):
    return pl.pallas_call(
        _tcb_body,
        grid=(_GRIDP,),
        in_specs=[
            pl.BlockSpec((_RBP, _H), lambda i: (i, 0)),
            pl.BlockSpec((_RBP, _H), lambda i: (i + _GRIDP, 0)),
            pl.BlockSpec((2, 1, _RBP, 2), lambda i: (0, i, 0, 0)),
        ],
        out_specs=pl.BlockSpec((_RBP, 2 * _H), lambda i: (i, 0)),
        out_shape=jax.ShapeDtypeStruct((_N // 2, 2 * _H), jnp.float32),
    )(h1, h1, deg_p)


def _dis128(deg_ref):
    deg = deg_ref[0, 0] + deg_ref[1, 0] + 1.0
    dist = jnp.broadcast_to(lax.rsqrt(deg[:, 0:1]), (_RBP, _H))
    disb = jnp.broadcast_to(lax.rsqrt(deg[:, 1:2]), (_RBP, _H))
    return jnp.concatenate([dist, disb], axis=1)


def _tc2_body(acc_ref, hp_ref, deg_ref, b_ref, w_ref, out_ref):
    acc = acc_ref[0] + acc_ref[1] + hp_ref[...]
    dis = _dis128(deg_ref)
    z = jnp.maximum(dis * acc + b_ref[...], 0.0)
    h2 = jnp.dot(z.astype(jnp.bfloat16), w_ref[...].astype(jnp.bfloat16),
                 preferred_element_type=jnp.float32)
    out_ref[...] = dis * h2


def _tc2(acc, h1p, deg_p, b1, W2):
    return pl.pallas_call(
        _tc2_body,
        grid=(_GRIDP,),
        in_specs=[
            pl.BlockSpec((2, _RBP, 2 * _H), lambda i: (0, i, 0)),
            pl.BlockSpec((_RBP, 2 * _H), lambda i: (i, 0)),
            pl.BlockSpec((2, 1, _RBP, 2), lambda i: (0, i, 0, 0)),
            pl.BlockSpec((1, 2 * _H), lambda i: (0, 0)),
            pl.BlockSpec((2 * _H, 2 * _H), lambda i: (0, 0)),
        ],
        out_specs=pl.BlockSpec((_RBP, 2 * _H), lambda i: (i, 0)),
        out_shape=jax.ShapeDtypeStruct((_N // 2, 2 * _H), jnp.float32),
    )(acc, h1p, deg_p, b1, W2)


def _tc3_body(acc_ref, hp_ref, deg_ref, b_ref, lw_ref, lb_ref, out_ref,
              pool_ref):
    i = pl.program_id(0)
    acc = acc_ref[0] + acc_ref[1] + hp_ref[...]
    dis = _dis128(deg_ref)
    z = jnp.maximum(dis * acc + b_ref[...], 0.0)
    part = jnp.sum(z, axis=0, keepdims=True)

    @pl.when(i == 0)
    def _():
        pool_ref[...] = part

    @pl.when(i > 0)
    def _():
        pool_ref[...] += part

    @pl.when(i == _GRIDP - 1)
    def _():
        pooled = (pool_ref[0:1, 0:_H] + pool_ref[0:1, _H:2 * _H]) * (1.0 / _N)
        out_ref[...] = jnp.dot(pooled.astype(jnp.bfloat16),
                               lw_ref[...].astype(jnp.bfloat16),
                               preferred_element_type=jnp.float32
                               ) + lb_ref[...]


def _tc3(acc, h2p, deg_p, b2, lin_W, lin_b):
    return pl.pallas_call(
        _tc3_body,
        grid=(_GRIDP,),
        in_specs=[
            pl.BlockSpec((2, _RBP, 2 * _H), lambda i: (0, i, 0)),
            pl.BlockSpec((_RBP, 2 * _H), lambda i: (i, 0)),
            pl.BlockSpec((2, 1, _RBP, 2), lambda i: (0, i, 0, 0)),
            pl.BlockSpec((1, 2 * _H), lambda i: (0, 0)),
            pl.BlockSpec((_H, 2), lambda i: (0, 0)),
            pl.BlockSpec((1, 2), lambda i: (0, 0)),
        ],
        out_specs=pl.BlockSpec((1, 2), lambda i: (0, 0)),
        out_shape=jax.ShapeDtypeStruct((1, 2), jnp.float32),
        scratch_shapes=[pltpu.VMEM((1, 2 * _H), jnp.float32)],
    )(acc, h2p, deg_p, b2, lin_W, lin_b)


def kernel(x, edge_index, batch, W1, b1, W2, b2, lin_W, lin_b):
    half = _N // 2
    pe = jnp.where(edge_index < half, edge_index * 2,
                   (edge_index - half) * 2 + 1)
    e3 = pe.reshape(2, _EROWS, _LANES)

    deg2 = _sc_degree(e3)
    h1 = _tca(x, W1)
    deg_p = deg2[:, :_N].reshape(2, _GRIDP, _RBP, 2)
    h1p = _tcb(h1, deg_p)
    acc1 = _sc_scatter(h1p.reshape(_N, _H), e3)
    acc1p = acc1.reshape(2, _N // 2, 2 * _H)
    b1p = jnp.concatenate([b1, b1]).reshape(1, 2 * _H)
    W2b = jax.scipy.linalg.block_diag(W2, W2)
    h2p = _tc2(acc1p, h1p, deg_p, b1p, W2b)
    acc2 = _sc_scatter(h2p.reshape(_N, _H), e3)
    acc2p = acc2.reshape(2, _N // 2, 2 * _H)
    b2p = jnp.concatenate([b2, b2]).reshape(1, 2 * _H)
    return _tc3(acc2p, h2p, deg_p, b2p, lin_W, lin_b.reshape(1, 2))
